# baseline (speedup 1.0000x reference)
import functools

import jax
import jax.numpy as jnp
from jax import lax
from jax.experimental import pallas as pl
from jax.experimental.pallas import tpu as pltpu
from jax.experimental.pallas import tpu_sc as plsc


def _table_body(ee_ref, ec_ref, cwt_ref, out_ref):
    out_ref[...] = ee_ref[...] + jnp.dot(
        ec_ref[...], cwt_ref[...], preferred_element_type=jnp.float32
    )


def _compute_table(element_embedding, electron_config, config_weight):
    Zmax, F = element_embedding.shape
    return pl.pallas_call(
        _table_body,
        out_shape=jax.ShapeDtypeStruct((Zmax, F), jnp.float32),
    )(element_embedding, electron_config, config_weight.T)


_NBUF = 7


@functools.lru_cache(maxsize=None)
def _make_gather(n_rows, n_chunks_w, ch, F, Zmax, NC, NS):
    mesh = plsc.VectorSubcoreMesh(core_axis_name="c", subcore_axis_name="s")
    nbuf = _NBUF

    @functools.partial(
        pl.kernel,
        mesh=mesh,
        out_type=jax.ShapeDtypeStruct((n_rows, F), jnp.float32),
        scratch_types=[
            pltpu.VMEM((n_chunks_w, ch), jnp.int32),
            pltpu.VMEM_SHARED((4 * Zmax, F), jnp.float32),
            pltpu.VMEM((nbuf, ch, F), jnp.float32),
        ]
        + [pltpu.SemaphoreType.DMA] * (2 * nbuf),
    )
    def gather(table_hbm, idx_hbm, out_hbm, idx_v, table_v, rows_v, *sems):
        gsem, wsem = sems[:nbuf], sems[nbuf:]
        wid = lax.axis_index("s") * NC + lax.axis_index("c")
        row0 = wid * n_chunks_w
        @pl.when(lax.axis_index("s") == 0)
        def _():
            for c in range(4):
                pltpu.sync_copy(table_hbm, table_v.at[pl.ds(c * Zmax, Zmax)])

        pltpu.sync_copy(idx_hbm.at[pl.ds(row0, n_chunks_w)], idx_v)
        plsc.subcore_barrier()

        n_rounds = n_chunks_w // nbuf
        tail = n_chunks_w % nbuf

        for b in range(nbuf):
            pltpu.async_copy(table_v.at[idx_v.at[b]], rows_v.at[b], gsem[b])

        def body(r, carry):
            for b in range(nbuf):
                j = r * nbuf + b
                pltpu.make_async_copy(
                    table_v.at[idx_v.at[j]], rows_v.at[b], gsem[b]
                ).wait()
                pltpu.async_copy(
                    rows_v.at[b], out_hbm.at[pl.ds((row0 + j) * ch, ch)], wsem[b]
                )
            for b in range(nbuf):
                j = r * nbuf + b
                pltpu.make_async_copy(
                    rows_v.at[b], out_hbm.at[pl.ds((row0 + j) * ch, ch)], wsem[b]
                ).wait()

                @pl.when(j + nbuf < n_chunks_w)
                def _():
                    jn = j + nbuf
                    pltpu.async_copy(
                        table_v.at[idx_v.at[jn]], rows_v.at[b], gsem[b]
                    )

            return carry

        lax.fori_loop(0, n_rounds, body, 0)

        for b in range(tail):
            j = n_rounds * nbuf + b
            pltpu.make_async_copy(
                table_v.at[idx_v.at[j]], rows_v.at[b], gsem[b]
            ).wait()
            pltpu.async_copy(
                rows_v.at[b], out_hbm.at[pl.ds((row0 + j) * ch, ch)], wsem[b]
            )
        for b in range(tail):
            j = n_rounds * nbuf + b
            pltpu.make_async_copy(
                rows_v.at[b], out_hbm.at[pl.ds((row0 + j) * ch, ch)], wsem[b]
            ).wait()

    return gather


def kernel(Z, element_embedding, config_weight, electron_config):
    B, N = Z.shape
    Zmax, F = element_embedding.shape
    table = _compute_table(element_embedding, electron_config, config_weight)

    info = plsc.get_sparse_core_info()
    NC, NS = info.num_cores, info.num_subcores
    NW = NC * NS

    ch = N
    n_chunks = B
    n_chunks_w = n_chunks // NW

    idx = Z.astype(jnp.int32)
    row_ids = jnp.arange(n_chunks, dtype=jnp.int32)
    copy_off = (((row_ids // n_chunks_w) // NC) % 4) * Zmax
    idx = idx + copy_off[:, None]
    out = _make_gather(B * N, n_chunks_w, ch, F, Zmax, NC, NS)(table, idx)
    return out.reshape(B, N, F)


# speedup vs baseline: 1.0994x; 1.0994x over previous
import functools

import jax
import jax.numpy as jnp
from jax import lax
from jax.experimental import pallas as pl
from jax.experimental.pallas import tpu as pltpu
from jax.experimental.pallas import tpu_sc as plsc


def _table_body(ee_ref, ec_ref, cwt_ref, out_ref):
    out_ref[...] = ee_ref[...] + jnp.dot(
        ec_ref[...], cwt_ref[...], preferred_element_type=jnp.float32
    )


def _compute_table(element_embedding, electron_config, config_weight):
    Zmax, F = element_embedding.shape
    return pl.pallas_call(
        _table_body,
        out_shape=jax.ShapeDtypeStruct((Zmax, F), jnp.float32),
    )(element_embedding, electron_config, config_weight.T)


_NBUF = 7


@functools.lru_cache(maxsize=None)
def _make_gather(n_rows, n_chunks_w, ch, F, Zmax, NC, NS):
    mesh = plsc.VectorSubcoreMesh(core_axis_name="c", subcore_axis_name="s")
    nbuf = _NBUF

    @functools.partial(
        pl.kernel,
        mesh=mesh,
        out_type=jax.ShapeDtypeStruct((n_rows, F), jnp.float32),
        scratch_types=[
            pltpu.VMEM((n_chunks_w, ch), jnp.int32),
            pltpu.VMEM_SHARED((Zmax, F), jnp.float32),
            pltpu.VMEM((nbuf, ch, F), jnp.float32),
        ]
        + [pltpu.SemaphoreType.DMA] * (2 * nbuf),
    )
    def gather(table_hbm, idx_hbm, out_hbm, idx_v, table_v, rows_v, *sems):
        gsem, wsem = sems[:nbuf], sems[nbuf:]
        wid = lax.axis_index("s") * NC + lax.axis_index("c")
        row0 = wid * n_chunks_w
        @pl.when(lax.axis_index("s") == 0)
        def _():
            pltpu.sync_copy(table_hbm, table_v)

        pltpu.sync_copy(idx_hbm.at[pl.ds(row0, n_chunks_w)], idx_v)
        plsc.subcore_barrier()

        n_rounds = n_chunks_w // nbuf
        tail = n_chunks_w % nbuf

        for b in range(nbuf):
            pltpu.async_copy(table_v.at[idx_v.at[b]], rows_v.at[b], gsem[b])

        def body(r, carry):
            for b in range(nbuf):
                j = r * nbuf + b
                pltpu.make_async_copy(
                    table_v.at[idx_v.at[j]], rows_v.at[b], gsem[b]
                ).wait()
                pltpu.async_copy(
                    rows_v.at[b], out_hbm.at[pl.ds((row0 + j) * ch, ch)], wsem[b]
                )
            for b in range(nbuf):
                j = r * nbuf + b
                pltpu.make_async_copy(
                    rows_v.at[b], out_hbm.at[pl.ds((row0 + j) * ch, ch)], wsem[b]
                ).wait()

                @pl.when(j + nbuf < n_chunks_w)
                def _():
                    jn = j + nbuf
                    pltpu.async_copy(
                        table_v.at[idx_v.at[jn]], rows_v.at[b], gsem[b]
                    )

            return carry

        lax.fori_loop(0, n_rounds, body, 0)

        for b in range(tail):
            j = n_rounds * nbuf + b
            pltpu.make_async_copy(
                table_v.at[idx_v.at[j]], rows_v.at[b], gsem[b]
            ).wait()
            pltpu.async_copy(
                rows_v.at[b], out_hbm.at[pl.ds((row0 + j) * ch, ch)], wsem[b]
            )
        for b in range(tail):
            j = n_rounds * nbuf + b
            pltpu.make_async_copy(
                rows_v.at[b], out_hbm.at[pl.ds((row0 + j) * ch, ch)], wsem[b]
            ).wait()

    return gather


def kernel(Z, element_embedding, config_weight, electron_config):
    B, N = Z.shape
    Zmax, F = element_embedding.shape
    table = _compute_table(element_embedding, electron_config, config_weight)

    info = plsc.get_sparse_core_info()
    NC, NS = info.num_cores, info.num_subcores
    NW = NC * NS

    ch = N
    n_chunks = B
    n_chunks_w = n_chunks // NW

    idx = Z.astype(jnp.int32)
    out = _make_gather(B * N, n_chunks_w, ch, F, Zmax, NC, NS)(table, idx)
    return out.reshape(B, N, F)
